# Initial kernel scaffold; baseline (speedup 1.0000x reference)
import jax
import jax.numpy as jnp
from jax.experimental import pallas as pl


def kernel(prefix, embedding_table):
    raise NotImplementedError("write your pallas kernel here")


# speedup vs baseline: 1.2050x; 1.2050x over previous
import jax
import jax.numpy as jnp
from jax import lax
from jax.experimental import pallas as pl
from jax.experimental.pallas import tpu as pltpu
from jax.experimental.pallas import tpu_sc as plsc

D = 14336
NC, NS = 2, 16
NW = NC * NS
B = 512
BPW = B // NW
CH = 4
NCHUNK = BPW // CH


def _body(idx_hbm, table_hbm, out_hbm, idx_v, buf0, buf1, g0, g1, w0, w1):
    wid = lax.axis_index("s") * NC + lax.axis_index("c")
    base = wid * BPW
    pltpu.sync_copy(idx_hbm.at[wid], idx_v)
    bufs = (buf0, buf1)
    gsems = (g0, g1)
    wsems = (w0, w1)

    def gather(j, b):
        return pltpu.make_async_copy(
            table_hbm.at[idx_v.at[j]], bufs[b], gsems[b])

    def write(j, b):
        return pltpu.make_async_copy(
            bufs[b], out_hbm.at[pl.ds(base + j * CH, CH)], wsems[b])

    gather(0, 0).start()
    gather(1, 1).start()
    for j in range(NCHUNK):
        b = j % 2
        gather(j, b).wait()
        write(j, b).start()
        if j + 2 < NCHUNK:
            write(j, b).wait()
            gather(j + 2, b).start()
    for j in (NCHUNK - 2, NCHUNK - 1):
        write(j, j % 2).wait()


_gather_call = pl.kernel(
    _body,
    out_type=jax.ShapeDtypeStruct((B, D), jnp.float32),
    mesh=plsc.VectorSubcoreMesh(core_axis_name="c", subcore_axis_name="s"),
    scratch_types=[
        pltpu.VMEM((NCHUNK, CH), jnp.int32),
        pltpu.VMEM((CH, D), jnp.float32),
        pltpu.VMEM((CH, D), jnp.float32),
        pltpu.SemaphoreType.DMA,
        pltpu.SemaphoreType.DMA,
        pltpu.SemaphoreType.DMA,
        pltpu.SemaphoreType.DMA,
    ],
)


def kernel(prefix, embedding_table):
    bsz, seq = prefix.shape
    idx = prefix.astype(jnp.int32).reshape(NW, NCHUNK, CH)
    out = _gather_call(idx, embedding_table)
    return out.reshape(bsz, seq, D)


# speedup vs baseline: 1.2244x; 1.0161x over previous
import jax
import jax.numpy as jnp
from jax import lax
from jax.experimental import pallas as pl
from jax.experimental.pallas import tpu as pltpu
from jax.experimental.pallas import tpu_sc as plsc

D = 14336
NC, NS = 2, 16
NW = NC * NS
B = 512
BPW = B // NW
CH = 2
NB = 4
NCHUNK = BPW // CH


def _body(idx_hbm, table_hbm, out_hbm, idx_v, buf0, buf1, buf2, buf3,
          g0, g1, g2, g3, w0, w1, w2, w3):
    wid = lax.axis_index("s") * NC + lax.axis_index("c")
    base = wid * BPW
    pltpu.sync_copy(idx_hbm.at[wid], idx_v)
    bufs = (buf0, buf1, buf2, buf3)
    gsems = (g0, g1, g2, g3)
    wsems = (w0, w1, w2, w3)

    def gather(j, b):
        return pltpu.make_async_copy(
            table_hbm.at[idx_v.at[j]], bufs[b], gsems[b])

    def write(j, b):
        return pltpu.make_async_copy(
            bufs[b], out_hbm.at[pl.ds(base + j * CH, CH)], wsems[b])

    for j in range(NB):
        gather(j, j).start()
    for j in range(NCHUNK):
        b = j % NB
        gather(j, b).wait()
        write(j, b).start()
        if j + NB < NCHUNK:
            write(j, b).wait()
            gather(j + NB, b).start()
    for j in range(NCHUNK - NB, NCHUNK):
        write(j, j % NB).wait()


_gather_call = pl.kernel(
    _body,
    out_type=jax.ShapeDtypeStruct((B, D), jnp.float32),
    mesh=plsc.VectorSubcoreMesh(core_axis_name="c", subcore_axis_name="s"),
    scratch_types=(
        [pltpu.VMEM((NCHUNK, CH), jnp.int32)]
        + [pltpu.VMEM((CH, D), jnp.float32)] * NB
        + [pltpu.SemaphoreType.DMA] * (2 * NB)
    ),
)


def kernel(prefix, embedding_table):
    bsz, seq = prefix.shape
    idx = prefix.astype(jnp.int32).reshape(NW, NCHUNK, CH)
    out = _gather_call(idx, embedding_table)
    return out.reshape(bsz, seq, D)
